# Initial kernel scaffold; baseline (speedup 1.0000x reference)
import jax
import jax.numpy as jnp
from jax.experimental import pallas as pl


def kernel(boxes, scores, labels):
    raise NotImplementedError("write your pallas kernel here")


# speedup vs baseline: 374.1414x; 374.1414x over previous
import jax
import jax.numpy as jnp
from jax import lax
from jax.experimental import pallas as pl
from jax.experimental.pallas import tpu as pltpu
from jax.experimental.pallas import tpu_sc as plsc

N = 5000
LANES = 16
NPAD = 5120
NCH = NPAD // LANES
NCLS = 80
HUMAN_IDX = 1
NMS_THRESH = 0.5
SCORE_THRESH = 0.2
KCAP = 15
TILES = 16
CPT = NCLS // TILES
NEGS = -3.0e38
DUMMY = 3.0e9
BIGI = 2**30


def _nms_body(vin, lbh, obh, osh, olh,
              vinf, vlb, midx, ms, t16f, t16i,
              gsc, gidx, heads_s, heads_i, rb, rs, rl,
              ssc, sidx):
    core = lax.axis_index("c")
    sub = lax.axis_index("s")
    lanes = lax.iota(jnp.int32, LANES)
    ones = lanes >= 0
    negs16 = jnp.full((LANES,), NEGS, jnp.float32)
    bigi16 = jnp.full((LANES,), BIGI, jnp.int32)

    @pl.when(core == 0)
    def _():
        pltpu.sync_copy(vin, vinf)
        pltpu.sync_copy(lbh, vlb)

        def mx_body(j, acc):
            a = jnp.maximum(vinf[2, pl.ds(j * LANES, LANES)],
                            vinf[3, pl.ds(j * LANES, LANES)])
            return jnp.maximum(acc, a)

        acc = lax.fori_loop(0, NCH, mx_body, negs16)
        maxc = jnp.max(acc) + jnp.float32(1.0)

        for k in range(CPT):
            c = sub * CPT + k
            off = c.astype(jnp.float32) * maxc

            def scan_body(j, cnt):
                lab16 = vlb[pl.ds(j * LANES, LANES)]
                sc16 = vinf[4, pl.ds(j * LANES, LANES)]
                m = (lab16 == c) & (sc16 >= SCORE_THRESH)
                idx16 = j * LANES + lanes
                plsc.store_compressed(midx.at[pl.ds(cnt, LANES)], idx16, mask=m)
                plsc.store_compressed(ms.at[pl.ds(cnt, LANES)], sc16, mask=m)
                return cnt + jnp.sum(m.astype(jnp.int32))

            cnt = lax.fori_loop(0, NCH, scan_body, jnp.int32(0))
            plsc.store_compressed(ms.at[pl.ds(cnt, LANES)], negs16, mask=ones)

            def cond(st):
                return (st[0] < cnt) & (st[1] < KCAP)

            def body(st):
                nproc, kcnt, kx1, ky1, kx2, ky2, kid, ksc = st
                nchk = (cnt + (LANES - 1)) >> 4

                def am_body(j, s):
                    bv, bp = s
                    v = ms[pl.ds(j * LANES, LANES)]
                    upd = v > bv
                    return jnp.where(upd, v, bv), jnp.where(upd, j, bp)

                bv, bp = lax.fori_loop(0, nchk, am_body,
                                       (negs16, jnp.zeros((LANES,), jnp.int32)))
                gmax = jnp.max(bv)
                posl = jnp.where(bv == gmax, bp * LANES + lanes, BIGI)
                pos = jnp.min(posl)
                posv = jnp.full((LANES,), pos, jnp.int32)
                plsc.store_scatter(ms, [posv], negs16, mask=lanes == 0)
                giv = plsc.load_gather(midx, [posv])
                c0 = jnp.zeros((LANES,), jnp.int32)
                cx1 = plsc.load_gather(vinf, [c0, giv]) + off
                cy1 = plsc.load_gather(vinf, [c0 + 1, giv]) + off
                cx2 = plsc.load_gather(vinf, [c0 + 2, giv]) + off
                cy2 = plsc.load_gather(vinf, [c0 + 3, giv]) + off
                w = jnp.maximum(jnp.minimum(kx2, cx2) - jnp.maximum(kx1, cx1), 0.0)
                h = jnp.maximum(jnp.minimum(ky2, cy2) - jnp.maximum(ky1, cy1), 0.0)
                inter = w * h
                ka = (kx2 - kx1) * (ky2 - ky1)
                ca = (cx2 - cx1) * (cy2 - cy1)
                iou = inter / jnp.maximum(ka + ca - inter, jnp.float32(1e-9))
                sup = jnp.max(iou) > NMS_THRESH
                addm = jnp.logical_and(jnp.logical_not(sup), lanes == kcnt)
                kx1 = jnp.where(addm, cx1, kx1)
                ky1 = jnp.where(addm, cy1, ky1)
                kx2 = jnp.where(addm, cx2, kx2)
                ky2 = jnp.where(addm, cy2, ky2)
                kid = jnp.where(addm, giv, kid)
                ksc = jnp.where(addm, gmax, ksc)
                kcnt = kcnt + jnp.where(sup, 0, 1).astype(jnp.int32)
                return (nproc + 1, kcnt, kx1, ky1, kx2, ky2, kid, ksc)

            dummy16 = jnp.full((LANES,), DUMMY, jnp.float32)
            st = lax.while_loop(cond, body,
                                (jnp.int32(0), jnp.int32(0),
                                 dummy16, dummy16, dummy16, dummy16,
                                 bigi16, negs16))
            kid, ksc = st[6], st[7]
            t16f[...] = ksc
            pltpu.sync_copy(t16f, ssc.at[pl.ds(c * LANES, LANES)])
            t16i[...] = kid
            pltpu.sync_copy(t16i, sidx.at[pl.ds(c * LANES, LANES)])

        plsc.subcore_barrier()

        @pl.when(sub == 0)
        def _():
            pltpu.sync_copy(ssc, gsc)
            pltpu.sync_copy(sidx, gidx)
            hs = gsc[pl.ds(HUMAN_IDX * LANES, LANES)]
            hi = gidx[pl.ds(HUMAN_IDX * LANES, LANES)]
            gsc[pl.ds(HUMAN_IDX * LANES, LANES)] = negs16
            for j in range(NCLS // LANES):
                rowv = (j * LANES + lanes) * LANES
                heads_s[pl.ds(j * LANES, LANES)] = plsc.load_gather(gsc, [rowv])
                heads_i[pl.ds(j * LANES, LANES)] = plsc.load_gather(gidx, [rowv])
            def ext_body(t, s):
                osc, oidx, ptrs0, ptrs1, ptrs2, ptrs3, ptrs4 = s

                def hb(j, hst):
                    bv, bi, bp = hst
                    v = heads_s[pl.ds(j * LANES, LANES)]
                    iv = heads_i[pl.ds(j * LANES, LANES)]
                    upd = (v > bv) | ((v == bv) & (iv < bi))
                    return (jnp.where(upd, v, bv), jnp.where(upd, iv, bi),
                            jnp.where(upd, j, bp))

                bv, bi, bp = lax.fori_loop(0, NCLS // LANES, hb,
                                           (negs16, bigi16,
                                            jnp.zeros((LANES,), jnp.int32)))
                gmax = jnp.max(bv)
                gidw = jnp.min(jnp.where(bv == gmax, bi, BIGI))
                cls = jnp.min(jnp.where((bv == gmax) & (bi == gidw),
                                        bp * LANES + lanes, BIGI))
                clsv = jnp.full((LANES,), cls, jnp.int32)
                pgrp = cls >> 4
                plane = cls & (LANES - 1)
                ptrs_all = [ptrs0, ptrs1, ptrs2, ptrs3, ptrs4]
                p = jnp.int32(0)
                for g in range(5):
                    p = p + jnp.where(pgrp == g,
                                      jnp.max(jnp.where(lanes == plane,
                                                        ptrs_all[g], 0)),
                                      0)
                p = p + 1
                new_ptrs = []
                for g in range(5):
                    updg = (pgrp == g) & (lanes == plane)
                    new_ptrs.append(jnp.where(updg, p, ptrs_all[g]))
                psafe = jnp.minimum(p, LANES - 1)
                hv = plsc.load_gather(gsc, [clsv * LANES + psafe])
                hiv = plsc.load_gather(gidx, [clsv * LANES + psafe])
                hv = jnp.where(p >= LANES, negs16, hv)
                plsc.store_scatter(heads_s, [clsv], hv, mask=lanes == 0)
                plsc.store_scatter(heads_i, [clsv], hiv, mask=lanes == 0)
                valid = gmax > jnp.float32(-1.0e37)
                osc = jnp.where((lanes == t) & valid, gmax, osc)
                oidx = jnp.where((lanes == t) & valid, gidw, oidx)
                return (osc, oidx, new_ptrs[0], new_ptrs[1], new_ptrs[2],
                        new_ptrs[3], new_ptrs[4])

            z16 = jnp.zeros((LANES,), jnp.int32)
            osc, oidx = lax.fori_loop(0, KCAP, ext_body,
                                      (negs16, bigi16, z16, z16, z16, z16,
                                       z16))[:2]

            heads_s[pl.ds(0, LANES)] = hs
            heads_i[pl.ds(0, LANES)] = hi
            heads_s[pl.ds(LANES, LANES)] = osc
            heads_i[pl.ds(LANES, LANES)] = oidx

            zf16 = jnp.zeros((LANES,), jnp.float32)
            for j in range(8):
                rb[pl.ds(j * LANES, LANES)] = zf16
            rs[pl.ds(0, LANES)] = zf16
            rs[pl.ds(LANES, LANES)] = zf16
            neg1 = jnp.full((LANES,), -1, jnp.int32)
            rl[pl.ds(0, LANES)] = neg1
            rl[pl.ds(LANES, LANES)] = neg1

            def mg_body(t, s):
                hp, op = s
                hpv = jnp.full((LANES,), hp, jnp.int32)
                opv = jnp.full((LANES,), op + LANES, jnp.int32)
                hv = plsc.load_gather(heads_s, [hpv])
                hiv = plsc.load_gather(heads_i, [hpv])
                ov = plsc.load_gather(heads_s, [opv])
                oiv = plsc.load_gather(heads_i, [opv])
                hvs = jnp.max(hv)
                ovs = jnp.max(ov)
                his = jnp.max(hiv)
                ois = jnp.max(oiv)
                takeh = (hvs > ovs) | ((hvs == ovs) & (his < ois))
                cs = jnp.where(takeh, hv, ov)
                ci = jnp.where(takeh, hiv, oiv)
                css = jnp.max(cs)
                valid = css > jnp.float32(-1.0e37)
                cis = jnp.where(valid, ci, jnp.zeros((LANES,), jnp.int32))
                c0 = jnp.zeros((LANES,), jnp.int32)
                m0 = (lanes == 0) & valid
                tv = jnp.full((LANES,), t, jnp.int32)
                bx1 = plsc.load_gather(vinf, [c0, cis])
                by1 = plsc.load_gather(vinf, [c0 + 1, cis])
                bx2 = plsc.load_gather(vinf, [c0 + 2, cis])
                by2 = plsc.load_gather(vinf, [c0 + 3, cis])
                lbv = plsc.load_gather(vlb, [cis])
                plsc.store_scatter(rb, [tv * 4], bx1, mask=m0)
                plsc.store_scatter(rb, [tv * 4 + 1], by1, mask=m0)
                plsc.store_scatter(rb, [tv * 4 + 2], bx2, mask=m0)
                plsc.store_scatter(rb, [tv * 4 + 3], by2, mask=m0)
                plsc.store_scatter(rs, [tv], cs, mask=m0)
                plsc.store_scatter(rl, [tv], lbv, mask=m0)
                adv = valid.astype(jnp.int32)
                hp = hp + jnp.where(takeh, adv, 0)
                op = op + jnp.where(takeh, 0, adv)
                return (hp, op)

            lax.fori_loop(0, 2 * KCAP, mg_body, (jnp.int32(0), jnp.int32(0)))

            pltpu.sync_copy(rb, obh)
            pltpu.sync_copy(rs, osh)
            pltpu.sync_copy(rl, olh)


_mesh = plsc.VectorSubcoreMesh(core_axis_name="c", subcore_axis_name="s",
                               num_cores=2, num_subcores=16)

_OUT_TYPE = [
    jax.ShapeDtypeStruct((128,), jnp.float32),
    jax.ShapeDtypeStruct((32,), jnp.float32),
    jax.ShapeDtypeStruct((32,), jnp.int32),
]

_SCRATCH_TYPES = [
        pltpu.VMEM((5, NPAD), jnp.float32),
        pltpu.VMEM((NPAD,), jnp.int32),
        pltpu.VMEM((NPAD + LANES,), jnp.int32),
        pltpu.VMEM((NPAD + LANES,), jnp.float32),
        pltpu.VMEM((LANES,), jnp.float32),
        pltpu.VMEM((LANES,), jnp.int32),
        pltpu.VMEM((NCLS * LANES,), jnp.float32),
        pltpu.VMEM((NCLS * LANES,), jnp.int32),
        pltpu.VMEM((NCLS,), jnp.float32),
        pltpu.VMEM((NCLS,), jnp.int32),
        pltpu.VMEM((128,), jnp.float32),
        pltpu.VMEM((32,), jnp.float32),
        pltpu.VMEM((32,), jnp.int32),
        pltpu.VMEM_SHARED((NCLS * LANES,), jnp.float32),
        pltpu.VMEM_SHARED((NCLS * LANES,), jnp.int32),
]

_sc_call = pl.kernel(
    _nms_body,
    out_type=_OUT_TYPE,
    mesh=_mesh,
    compiler_params=pltpu.CompilerParams(needs_layout_passes=False),
    scratch_types=_SCRATCH_TYPES,
)


@jax.jit
def kernel(boxes, scores, labels):
    pad = NPAD - N
    zf = jnp.zeros((pad,), jnp.float32)
    vin = jnp.stack([
        jnp.concatenate([boxes[:, 0], zf]),
        jnp.concatenate([boxes[:, 1], zf]),
        jnp.concatenate([boxes[:, 2], zf]),
        jnp.concatenate([boxes[:, 3], zf]),
        jnp.concatenate([scores, jnp.full((pad,), -1.0, jnp.float32)]),
    ])
    lb = jnp.concatenate([labels, jnp.full((pad,), -1, jnp.int32)])
    obf, osf, olf = _sc_call(vin, lb)
    return obf[:120].reshape(30, 4), osf[:30], olf[:30]


# speedup vs baseline: 472.8333x; 1.2638x over previous
import jax
import jax.numpy as jnp
from jax import lax
from jax.experimental import pallas as pl
from jax.experimental.pallas import tpu as pltpu
from jax.experimental.pallas import tpu_sc as plsc

N = 5000
LANES = 16
NPAD = 5120
NCH = NPAD // LANES
NCLS = 80
HUMAN_IDX = 1
NMS_THRESH = 0.5
SCORE_THRESH = 0.2
KCAP = 15
TILES = 16
CPT = NCLS // TILES
NEGS = -3.0e38
DUMMY = 3.0e9
BIGI = 2**30


def _nms_body(vin, lbh, obh, osh, olh,
              vinf, vlb, rmidx, rmlab, rms, midx, ms, t16f, t16i,
              gsc, gidx, heads_s, heads_i, rb, rs, rl,
              ssc, sidx):
    core = lax.axis_index("c")
    sub = lax.axis_index("s")
    lanes = lax.iota(jnp.int32, LANES)
    ones = lanes >= 0
    negs16 = jnp.full((LANES,), NEGS, jnp.float32)
    bigi16 = jnp.full((LANES,), BIGI, jnp.int32)

    @pl.when(core == 0)
    def _():
        pltpu.sync_copy(vin, vinf)
        pltpu.sync_copy(lbh, vlb)

        def mx_body(j, acc):
            a = jnp.maximum(vinf[2, pl.ds(j * LANES, LANES)],
                            vinf[3, pl.ds(j * LANES, LANES)])
            return jnp.maximum(acc, a)

        acc = lax.fori_loop(0, NCH, mx_body, negs16)
        maxc = jnp.max(acc) + jnp.float32(1.0)

        lo = sub * CPT

        def rscan(j, cnt):
            lab16 = vlb[pl.ds(j * LANES, LANES)]
            sc16 = vinf[4, pl.ds(j * LANES, LANES)]
            m = (lab16 >= lo) & (lab16 < lo + CPT) & (sc16 >= SCORE_THRESH)
            idx16 = j * LANES + lanes
            plsc.store_compressed(rmidx.at[pl.ds(cnt, LANES)], idx16, mask=m)
            plsc.store_compressed(rmlab.at[pl.ds(cnt, LANES)], lab16, mask=m)
            plsc.store_compressed(rms.at[pl.ds(cnt, LANES)], sc16, mask=m)
            return cnt + jnp.sum(m.astype(jnp.int32))

        rcnt = lax.fori_loop(0, NCH, rscan, jnp.int32(0))
        plsc.store_compressed(rmlab.at[pl.ds(rcnt, LANES)],
                              jnp.full((LANES,), -1, jnp.int32), mask=ones)
        rch = (rcnt + (LANES - 1)) >> 4

        for k in range(CPT):
            c = lo + k
            off = c.astype(jnp.float32) * maxc

            def scan_body(j, cnt):
                lab16 = rmlab[pl.ds(j * LANES, LANES)]
                m = lab16 == c
                plsc.store_compressed(midx.at[pl.ds(cnt, LANES)],
                                      rmidx[pl.ds(j * LANES, LANES)], mask=m)
                plsc.store_compressed(ms.at[pl.ds(cnt, LANES)],
                                      rms[pl.ds(j * LANES, LANES)], mask=m)
                return cnt + jnp.sum(m.astype(jnp.int32))

            cnt = lax.fori_loop(0, rch, scan_body, jnp.int32(0))
            plsc.store_compressed(ms.at[pl.ds(cnt, LANES)], negs16, mask=ones)

            def cond(st):
                return (st[0] < cnt) & (st[1] < KCAP)

            def body(st):
                nproc, kcnt, kx1, ky1, kx2, ky2, kid, ksc = st
                nchk = (cnt + (LANES - 1)) >> 4

                def am_body(j, s):
                    bv, bp = s
                    v = ms[pl.ds(j * LANES, LANES)]
                    upd = v > bv
                    return jnp.where(upd, v, bv), jnp.where(upd, j, bp)

                bv, bp = lax.fori_loop(0, nchk, am_body,
                                       (negs16, jnp.zeros((LANES,), jnp.int32)))
                gmax = jnp.max(bv)
                posl = jnp.where(bv == gmax, bp * LANES + lanes, BIGI)
                pos = jnp.min(posl)
                posv = jnp.full((LANES,), pos, jnp.int32)
                plsc.store_scatter(ms, [posv], negs16, mask=lanes == 0)
                giv = plsc.load_gather(midx, [posv])
                c0 = jnp.zeros((LANES,), jnp.int32)
                cx1 = plsc.load_gather(vinf, [c0, giv]) + off
                cy1 = plsc.load_gather(vinf, [c0 + 1, giv]) + off
                cx2 = plsc.load_gather(vinf, [c0 + 2, giv]) + off
                cy2 = plsc.load_gather(vinf, [c0 + 3, giv]) + off
                w = jnp.maximum(jnp.minimum(kx2, cx2) - jnp.maximum(kx1, cx1), 0.0)
                h = jnp.maximum(jnp.minimum(ky2, cy2) - jnp.maximum(ky1, cy1), 0.0)
                inter = w * h
                ka = (kx2 - kx1) * (ky2 - ky1)
                ca = (cx2 - cx1) * (cy2 - cy1)
                iou = inter / jnp.maximum(ka + ca - inter, jnp.float32(1e-9))
                sup = jnp.max(iou) > NMS_THRESH
                addm = jnp.logical_and(jnp.logical_not(sup), lanes == kcnt)
                kx1 = jnp.where(addm, cx1, kx1)
                ky1 = jnp.where(addm, cy1, ky1)
                kx2 = jnp.where(addm, cx2, kx2)
                ky2 = jnp.where(addm, cy2, ky2)
                kid = jnp.where(addm, giv, kid)
                ksc = jnp.where(addm, gmax, ksc)
                kcnt = kcnt + jnp.where(sup, 0, 1).astype(jnp.int32)
                return (nproc + 1, kcnt, kx1, ky1, kx2, ky2, kid, ksc)

            dummy16 = jnp.full((LANES,), DUMMY, jnp.float32)
            st = lax.while_loop(cond, body,
                                (jnp.int32(0), jnp.int32(0),
                                 dummy16, dummy16, dummy16, dummy16,
                                 bigi16, negs16))
            kid, ksc = st[6], st[7]
            t16f[...] = ksc
            pltpu.sync_copy(t16f, ssc.at[pl.ds(c * LANES, LANES)])
            t16i[...] = kid
            pltpu.sync_copy(t16i, sidx.at[pl.ds(c * LANES, LANES)])

        plsc.subcore_barrier()

        @pl.when(sub == 0)
        def _():
            pltpu.sync_copy(ssc, gsc)
            pltpu.sync_copy(sidx, gidx)
            hs = gsc[pl.ds(HUMAN_IDX * LANES, LANES)]
            hi = gidx[pl.ds(HUMAN_IDX * LANES, LANES)]
            gsc[pl.ds(HUMAN_IDX * LANES, LANES)] = negs16
            for j in range(NCLS // LANES):
                rowv = (j * LANES + lanes) * LANES
                heads_s[pl.ds(j * LANES, LANES)] = plsc.load_gather(gsc, [rowv])
                heads_i[pl.ds(j * LANES, LANES)] = plsc.load_gather(gidx, [rowv])
            def ext_body(t, s):
                osc, oidx, ptrs0, ptrs1, ptrs2, ptrs3, ptrs4 = s

                def hb(j, hst):
                    bv, bi, bp = hst
                    v = heads_s[pl.ds(j * LANES, LANES)]
                    iv = heads_i[pl.ds(j * LANES, LANES)]
                    upd = (v > bv) | ((v == bv) & (iv < bi))
                    return (jnp.where(upd, v, bv), jnp.where(upd, iv, bi),
                            jnp.where(upd, j, bp))

                bv, bi, bp = lax.fori_loop(0, NCLS // LANES, hb,
                                           (negs16, bigi16,
                                            jnp.zeros((LANES,), jnp.int32)))
                gmax = jnp.max(bv)
                gidw = jnp.min(jnp.where(bv == gmax, bi, BIGI))
                cls = jnp.min(jnp.where((bv == gmax) & (bi == gidw),
                                        bp * LANES + lanes, BIGI))
                clsv = jnp.full((LANES,), cls, jnp.int32)
                pgrp = cls >> 4
                plane = cls & (LANES - 1)
                ptrs_all = [ptrs0, ptrs1, ptrs2, ptrs3, ptrs4]
                p = jnp.int32(0)
                for g in range(5):
                    p = p + jnp.where(pgrp == g,
                                      jnp.max(jnp.where(lanes == plane,
                                                        ptrs_all[g], 0)),
                                      0)
                p = p + 1
                new_ptrs = []
                for g in range(5):
                    updg = (pgrp == g) & (lanes == plane)
                    new_ptrs.append(jnp.where(updg, p, ptrs_all[g]))
                psafe = jnp.minimum(p, LANES - 1)
                hv = plsc.load_gather(gsc, [clsv * LANES + psafe])
                hiv = plsc.load_gather(gidx, [clsv * LANES + psafe])
                hv = jnp.where(p >= LANES, negs16, hv)
                plsc.store_scatter(heads_s, [clsv], hv, mask=lanes == 0)
                plsc.store_scatter(heads_i, [clsv], hiv, mask=lanes == 0)
                valid = gmax > jnp.float32(-1.0e37)
                osc = jnp.where((lanes == t) & valid, gmax, osc)
                oidx = jnp.where((lanes == t) & valid, gidw, oidx)
                return (osc, oidx, new_ptrs[0], new_ptrs[1], new_ptrs[2],
                        new_ptrs[3], new_ptrs[4])

            z16 = jnp.zeros((LANES,), jnp.int32)
            osc, oidx = lax.fori_loop(0, KCAP, ext_body,
                                      (negs16, bigi16, z16, z16, z16, z16,
                                       z16))[:2]

            heads_s[pl.ds(0, LANES)] = hs
            heads_i[pl.ds(0, LANES)] = hi
            heads_s[pl.ds(LANES, LANES)] = osc
            heads_i[pl.ds(LANES, LANES)] = oidx

            zf16 = jnp.zeros((LANES,), jnp.float32)
            for j in range(8):
                rb[pl.ds(j * LANES, LANES)] = zf16
            rs[pl.ds(0, LANES)] = zf16
            rs[pl.ds(LANES, LANES)] = zf16
            neg1 = jnp.full((LANES,), -1, jnp.int32)
            rl[pl.ds(0, LANES)] = neg1
            rl[pl.ds(LANES, LANES)] = neg1

            def mg_body(t, s):
                hp, op = s
                hpv = jnp.full((LANES,), hp, jnp.int32)
                opv = jnp.full((LANES,), op + LANES, jnp.int32)
                hv = plsc.load_gather(heads_s, [hpv])
                hiv = plsc.load_gather(heads_i, [hpv])
                ov = plsc.load_gather(heads_s, [opv])
                oiv = plsc.load_gather(heads_i, [opv])
                hvs = jnp.max(hv)
                ovs = jnp.max(ov)
                his = jnp.max(hiv)
                ois = jnp.max(oiv)
                takeh = (hvs > ovs) | ((hvs == ovs) & (his < ois))
                cs = jnp.where(takeh, hv, ov)
                ci = jnp.where(takeh, hiv, oiv)
                css = jnp.max(cs)
                valid = css > jnp.float32(-1.0e37)
                cis = jnp.where(valid, ci, jnp.zeros((LANES,), jnp.int32))
                c0 = jnp.zeros((LANES,), jnp.int32)
                m0 = (lanes == 0) & valid
                tv = jnp.full((LANES,), t, jnp.int32)
                bx1 = plsc.load_gather(vinf, [c0, cis])
                by1 = plsc.load_gather(vinf, [c0 + 1, cis])
                bx2 = plsc.load_gather(vinf, [c0 + 2, cis])
                by2 = plsc.load_gather(vinf, [c0 + 3, cis])
                lbv = plsc.load_gather(vlb, [cis])
                plsc.store_scatter(rb, [tv * 4], bx1, mask=m0)
                plsc.store_scatter(rb, [tv * 4 + 1], by1, mask=m0)
                plsc.store_scatter(rb, [tv * 4 + 2], bx2, mask=m0)
                plsc.store_scatter(rb, [tv * 4 + 3], by2, mask=m0)
                plsc.store_scatter(rs, [tv], cs, mask=m0)
                plsc.store_scatter(rl, [tv], lbv, mask=m0)
                adv = valid.astype(jnp.int32)
                hp = hp + jnp.where(takeh, adv, 0)
                op = op + jnp.where(takeh, 0, adv)
                return (hp, op)

            lax.fori_loop(0, 2 * KCAP, mg_body, (jnp.int32(0), jnp.int32(0)))

            pltpu.sync_copy(rb, obh)
            pltpu.sync_copy(rs, osh)
            pltpu.sync_copy(rl, olh)


_mesh = plsc.VectorSubcoreMesh(core_axis_name="c", subcore_axis_name="s",
                               num_cores=2, num_subcores=16)

_OUT_TYPE = [
    jax.ShapeDtypeStruct((128,), jnp.float32),
    jax.ShapeDtypeStruct((32,), jnp.float32),
    jax.ShapeDtypeStruct((32,), jnp.int32),
]

_SCRATCH_TYPES = [
        pltpu.VMEM((5, NPAD), jnp.float32),
        pltpu.VMEM((NPAD,), jnp.int32),
        pltpu.VMEM((NPAD + LANES,), jnp.int32),
        pltpu.VMEM((NPAD + LANES,), jnp.int32),
        pltpu.VMEM((NPAD + LANES,), jnp.float32),
        pltpu.VMEM((NPAD + LANES,), jnp.int32),
        pltpu.VMEM((NPAD + LANES,), jnp.float32),
        pltpu.VMEM((LANES,), jnp.float32),
        pltpu.VMEM((LANES,), jnp.int32),
        pltpu.VMEM((NCLS * LANES,), jnp.float32),
        pltpu.VMEM((NCLS * LANES,), jnp.int32),
        pltpu.VMEM((NCLS,), jnp.float32),
        pltpu.VMEM((NCLS,), jnp.int32),
        pltpu.VMEM((128,), jnp.float32),
        pltpu.VMEM((32,), jnp.float32),
        pltpu.VMEM((32,), jnp.int32),
        pltpu.VMEM_SHARED((NCLS * LANES,), jnp.float32),
        pltpu.VMEM_SHARED((NCLS * LANES,), jnp.int32),
]

_sc_call = pl.kernel(
    _nms_body,
    out_type=_OUT_TYPE,
    mesh=_mesh,
    compiler_params=pltpu.CompilerParams(needs_layout_passes=False),
    scratch_types=_SCRATCH_TYPES,
)


@jax.jit
def kernel(boxes, scores, labels):
    pad = NPAD - N
    zf = jnp.zeros((pad,), jnp.float32)
    vin = jnp.stack([
        jnp.concatenate([boxes[:, 0], zf]),
        jnp.concatenate([boxes[:, 1], zf]),
        jnp.concatenate([boxes[:, 2], zf]),
        jnp.concatenate([boxes[:, 3], zf]),
        jnp.concatenate([scores, jnp.full((pad,), -1.0, jnp.float32)]),
    ])
    lb = jnp.concatenate([labels, jnp.full((pad,), -1, jnp.int32)])
    obf, osf, olf = _sc_call(vin, lb)
    return obf[:120].reshape(30, 4), osf[:30], olf[:30]


# speedup vs baseline: 502.3989x; 1.0625x over previous
import jax
import jax.numpy as jnp
from jax import lax
from jax.experimental import pallas as pl
from jax.experimental.pallas import tpu as pltpu
from jax.experimental.pallas import tpu_sc as plsc

N = 5000
LANES = 16
NPAD = 5120
NCH = NPAD // LANES
NCLS = 80
HUMAN_IDX = 1
NMS_THRESH = 0.5
SCORE_THRESH = 0.2
KCAP = 15
TILES = 16
CPT = NCLS // TILES
NEGS = -3.0e38
DUMMY = 3.0e9
BIGI = 2**30


def _nms_body(vin, lbh, obh, osh, olh,
              vinf, vlb, rmidx, rmlab, rms, midx, ms, t16f, t16i,
              gsc, gidx, heads_s, heads_i, ptrv, rb, rs, rl,
              sem1, sem2, ssc, sidx):
    core = lax.axis_index("c")
    sub = lax.axis_index("s")
    lanes = lax.iota(jnp.int32, LANES)
    ones = lanes >= 0
    negs16 = jnp.full((LANES,), NEGS, jnp.float32)
    bigi16 = jnp.full((LANES,), BIGI, jnp.int32)

    @pl.when(core == 0)
    def _():
        cp1 = pltpu.async_copy(vin, vinf, sem1)
        cp2 = pltpu.async_copy(lbh, vlb, sem2)
        cp1.wait()
        cp2.wait()

        def mx_body(j, acc):
            a = jnp.maximum(vinf[2, pl.ds(j * 2 * LANES, LANES)],
                            vinf[3, pl.ds(j * 2 * LANES, LANES)])
            b = jnp.maximum(vinf[2, pl.ds(j * 2 * LANES + LANES, LANES)],
                            vinf[3, pl.ds(j * 2 * LANES + LANES, LANES)])
            return jnp.maximum(acc, jnp.maximum(a, b))

        acc = lax.fori_loop(0, NCH // 2, mx_body, negs16)
        maxc = jnp.max(acc) + jnp.float32(1.0)

        lo = sub * CPT

        def rscan(j, cnt):
            lab16 = vlb[pl.ds(j * LANES, LANES)]
            sc16 = vinf[4, pl.ds(j * LANES, LANES)]
            m = (lab16 >= lo) & (lab16 < lo + CPT) & (sc16 >= SCORE_THRESH)
            idx16 = j * LANES + lanes
            plsc.store_compressed(rmidx.at[pl.ds(cnt, LANES)], idx16, mask=m)
            plsc.store_compressed(rmlab.at[pl.ds(cnt, LANES)], lab16, mask=m)
            plsc.store_compressed(rms.at[pl.ds(cnt, LANES)], sc16, mask=m)
            return cnt + plsc.all_reduce_population_count(m)[0]

        rcnt = lax.fori_loop(0, NCH, rscan, jnp.int32(0))
        plsc.store_compressed(rmlab.at[pl.ds(rcnt, LANES)],
                              jnp.full((LANES,), -1, jnp.int32), mask=ones)
        rch = (rcnt + (LANES - 1)) >> 4

        for k in range(CPT):
            c = lo + k
            off = c.astype(jnp.float32) * maxc

            def scan_body(j, cnt):
                lab16 = rmlab[pl.ds(j * LANES, LANES)]
                m = lab16 == c
                plsc.store_compressed(midx.at[pl.ds(cnt, LANES)],
                                      rmidx[pl.ds(j * LANES, LANES)], mask=m)
                plsc.store_compressed(ms.at[pl.ds(cnt, LANES)],
                                      rms[pl.ds(j * LANES, LANES)], mask=m)
                return cnt + plsc.all_reduce_population_count(m)[0]

            cnt = lax.fori_loop(0, rch, scan_body, jnp.int32(0))
            plsc.store_compressed(ms.at[pl.ds(cnt, LANES)], negs16, mask=ones)

            def cond(st):
                return (st[0] < cnt) & (st[1] < KCAP)

            def body(st):
                nproc, kcnt, kx1, ky1, kx2, ky2, kid, ksc = st
                nchk = (cnt + (LANES - 1)) >> 4

                def am_body(j, s):
                    bv, bp = s
                    v = ms[pl.ds(j * LANES, LANES)]
                    upd = v > bv
                    return jnp.where(upd, v, bv), jnp.where(upd, j, bp)

                bv, bp = lax.fori_loop(0, nchk, am_body,
                                       (negs16, jnp.zeros((LANES,), jnp.int32)))
                gmax = jnp.max(bv)
                posl = jnp.where(bv == gmax, bp * LANES + lanes, BIGI)
                pos = jnp.min(posl)
                posv = jnp.full((LANES,), pos, jnp.int32)
                plsc.store_scatter(ms, [posv], negs16, mask=lanes == 0)
                giv = plsc.load_gather(midx, [posv])
                c0 = jnp.zeros((LANES,), jnp.int32)
                cx1 = plsc.load_gather(vinf, [c0, giv]) + off
                cy1 = plsc.load_gather(vinf, [c0 + 1, giv]) + off
                cx2 = plsc.load_gather(vinf, [c0 + 2, giv]) + off
                cy2 = plsc.load_gather(vinf, [c0 + 3, giv]) + off
                w = jnp.maximum(jnp.minimum(kx2, cx2) - jnp.maximum(kx1, cx1), 0.0)
                h = jnp.maximum(jnp.minimum(ky2, cy2) - jnp.maximum(ky1, cy1), 0.0)
                inter = w * h
                ka = (kx2 - kx1) * (ky2 - ky1)
                ca = (cx2 - cx1) * (cy2 - cy1)
                iou = inter / jnp.maximum(ka + ca - inter, jnp.float32(1e-9))
                sup = plsc.all_reduce_population_count(iou > NMS_THRESH)[0] > 0
                addm = jnp.logical_and(jnp.logical_not(sup), lanes == kcnt)
                kx1 = jnp.where(addm, cx1, kx1)
                ky1 = jnp.where(addm, cy1, ky1)
                kx2 = jnp.where(addm, cx2, kx2)
                ky2 = jnp.where(addm, cy2, ky2)
                kid = jnp.where(addm, giv, kid)
                ksc = jnp.where(addm, gmax, ksc)
                kcnt = kcnt + jnp.where(sup, 0, 1).astype(jnp.int32)
                return (nproc + 1, kcnt, kx1, ky1, kx2, ky2, kid, ksc)

            dummy16 = jnp.full((LANES,), DUMMY, jnp.float32)
            st = lax.while_loop(cond, body,
                                (jnp.int32(0), jnp.int32(0),
                                 dummy16, dummy16, dummy16, dummy16,
                                 bigi16, negs16))
            kid, ksc = st[6], st[7]
            t16f[...] = ksc
            pltpu.sync_copy(t16f, ssc.at[pl.ds(c * LANES, LANES)])
            t16i[...] = kid
            pltpu.sync_copy(t16i, sidx.at[pl.ds(c * LANES, LANES)])

        plsc.subcore_barrier()

        @pl.when(sub == 0)
        def _():
            pltpu.sync_copy(ssc, gsc)
            pltpu.sync_copy(sidx, gidx)
            hs = gsc[pl.ds(HUMAN_IDX * LANES, LANES)]
            hi = gidx[pl.ds(HUMAN_IDX * LANES, LANES)]
            gsc[pl.ds(HUMAN_IDX * LANES, LANES)] = negs16
            for j in range(NCLS // LANES):
                rowv = (j * LANES + lanes) * LANES
                heads_s[pl.ds(j * LANES, LANES)] = plsc.load_gather(gsc, [rowv])
                heads_i[pl.ds(j * LANES, LANES)] = plsc.load_gather(gidx, [rowv])
            one16 = jnp.full((LANES,), 1, jnp.int32)
            for j in range(NCLS // LANES):
                ptrv[pl.ds(j * LANES, LANES)] = one16

            def ext_body(t, s):
                osc, oidx = s

                def hb(j, hst):
                    bv, bi, bp = hst
                    v = heads_s[pl.ds(j * LANES, LANES)]
                    iv = heads_i[pl.ds(j * LANES, LANES)]
                    upd = (v > bv) | ((v == bv) & (iv < bi))
                    return (jnp.where(upd, v, bv), jnp.where(upd, iv, bi),
                            jnp.where(upd, j, bp))

                bv, bi, bp = lax.fori_loop(0, NCLS // LANES, hb,
                                           (negs16, bigi16,
                                            jnp.zeros((LANES,), jnp.int32)))
                gmax = jnp.max(bv)
                gidw = jnp.min(jnp.where(bv == gmax, bi, BIGI))
                cls = jnp.min(jnp.where((bv == gmax) & (bi == gidw),
                                        bp * LANES + lanes, BIGI))
                clsv = jnp.full((LANES,), cls, jnp.int32)
                p = plsc.load_gather(ptrv, [clsv])
                plsc.store_scatter(ptrv, [clsv], p + 1, mask=lanes == 0)
                psafe = jnp.minimum(p, LANES - 1)
                hv = plsc.load_gather(gsc, [clsv * LANES + psafe])
                hiv = plsc.load_gather(gidx, [clsv * LANES + psafe])
                hv = jnp.where(p >= LANES, negs16, hv)
                plsc.store_scatter(heads_s, [clsv], hv, mask=lanes == 0)
                plsc.store_scatter(heads_i, [clsv], hiv, mask=lanes == 0)
                valid = gmax > jnp.float32(-1.0e37)
                osc = jnp.where((lanes == t) & valid, gmax, osc)
                oidx = jnp.where((lanes == t) & valid, gidw, oidx)
                return (osc, oidx)

            osc, oidx = lax.fori_loop(0, KCAP, ext_body, (negs16, bigi16))

            heads_s[pl.ds(0, LANES)] = hs
            heads_i[pl.ds(0, LANES)] = hi
            heads_s[pl.ds(LANES, LANES)] = osc
            heads_i[pl.ds(LANES, LANES)] = oidx

            zf16 = jnp.zeros((LANES,), jnp.float32)
            for j in range(8):
                rb[pl.ds(j * LANES, LANES)] = zf16
            rs[pl.ds(0, LANES)] = zf16
            rs[pl.ds(LANES, LANES)] = zf16
            neg1 = jnp.full((LANES,), -1, jnp.int32)
            rl[pl.ds(0, LANES)] = neg1
            rl[pl.ds(LANES, LANES)] = neg1

            def mg_body(t, s):
                hp, op = s
                hpv = jnp.full((LANES,), hp, jnp.int32)
                opv = jnp.full((LANES,), op + LANES, jnp.int32)
                hv = plsc.load_gather(heads_s, [hpv])
                hiv = plsc.load_gather(heads_i, [hpv])
                ov = plsc.load_gather(heads_s, [opv])
                oiv = plsc.load_gather(heads_i, [opv])
                hvs = hv[0]
                ovs = ov[0]
                his = hiv[0]
                ois = oiv[0]
                takeh = (hvs > ovs) | ((hvs == ovs) & (his < ois))
                cs = jnp.where(takeh, hv, ov)
                ci = jnp.where(takeh, hiv, oiv)
                valid = cs[0] > jnp.float32(-1.0e37)
                cis = jnp.where(valid, ci, jnp.zeros((LANES,), jnp.int32))
                c0 = jnp.zeros((LANES,), jnp.int32)
                m0 = (lanes == 0) & valid
                tv = jnp.full((LANES,), t, jnp.int32)
                bx1 = plsc.load_gather(vinf, [c0, cis])
                by1 = plsc.load_gather(vinf, [c0 + 1, cis])
                bx2 = plsc.load_gather(vinf, [c0 + 2, cis])
                by2 = plsc.load_gather(vinf, [c0 + 3, cis])
                lbv = plsc.load_gather(vlb, [cis])
                plsc.store_scatter(rb, [tv * 4], bx1, mask=m0)
                plsc.store_scatter(rb, [tv * 4 + 1], by1, mask=m0)
                plsc.store_scatter(rb, [tv * 4 + 2], bx2, mask=m0)
                plsc.store_scatter(rb, [tv * 4 + 3], by2, mask=m0)
                plsc.store_scatter(rs, [tv], cs, mask=m0)
                plsc.store_scatter(rl, [tv], lbv, mask=m0)
                adv = valid.astype(jnp.int32)
                hp = hp + jnp.where(takeh, adv, 0)
                op = op + jnp.where(takeh, 0, adv)
                return (hp, op)

            lax.fori_loop(0, 2 * KCAP, mg_body, (jnp.int32(0), jnp.int32(0)))

            pltpu.sync_copy(rb, obh)
            pltpu.sync_copy(rs, osh)
            pltpu.sync_copy(rl, olh)


_mesh = plsc.VectorSubcoreMesh(core_axis_name="c", subcore_axis_name="s",
                               num_cores=2, num_subcores=16)

_OUT_TYPE = [
    jax.ShapeDtypeStruct((128,), jnp.float32),
    jax.ShapeDtypeStruct((32,), jnp.float32),
    jax.ShapeDtypeStruct((32,), jnp.int32),
]

_SCRATCH_TYPES = [
        pltpu.VMEM((5, NPAD), jnp.float32),
        pltpu.VMEM((NPAD,), jnp.int32),
        pltpu.VMEM((NPAD + LANES,), jnp.int32),
        pltpu.VMEM((NPAD + LANES,), jnp.int32),
        pltpu.VMEM((NPAD + LANES,), jnp.float32),
        pltpu.VMEM((NPAD + LANES,), jnp.int32),
        pltpu.VMEM((NPAD + LANES,), jnp.float32),
        pltpu.VMEM((LANES,), jnp.float32),
        pltpu.VMEM((LANES,), jnp.int32),
        pltpu.VMEM((NCLS * LANES,), jnp.float32),
        pltpu.VMEM((NCLS * LANES,), jnp.int32),
        pltpu.VMEM((NCLS,), jnp.float32),
        pltpu.VMEM((NCLS,), jnp.int32),
        pltpu.VMEM((NCLS,), jnp.int32),
        pltpu.VMEM((128,), jnp.float32),
        pltpu.VMEM((32,), jnp.float32),
        pltpu.VMEM((32,), jnp.int32),
        pltpu.SemaphoreType.DMA,
        pltpu.SemaphoreType.DMA,
        pltpu.VMEM_SHARED((NCLS * LANES,), jnp.float32),
        pltpu.VMEM_SHARED((NCLS * LANES,), jnp.int32),
]

_sc_call = pl.kernel(
    _nms_body,
    out_type=_OUT_TYPE,
    mesh=_mesh,
    compiler_params=pltpu.CompilerParams(needs_layout_passes=False),
    scratch_types=_SCRATCH_TYPES,
)


@jax.jit
def kernel(boxes, scores, labels):
    pad = NPAD - N
    zf = jnp.zeros((pad,), jnp.float32)
    vin = jnp.stack([
        jnp.concatenate([boxes[:, 0], zf]),
        jnp.concatenate([boxes[:, 1], zf]),
        jnp.concatenate([boxes[:, 2], zf]),
        jnp.concatenate([boxes[:, 3], zf]),
        jnp.concatenate([scores, jnp.full((pad,), -1.0, jnp.float32)]),
    ])
    lb = jnp.concatenate([labels, jnp.full((pad,), -1, jnp.int32)])
    obf, osf, olf = _sc_call(vin, lb)
    return obf[:120].reshape(30, 4), osf[:30], olf[:30]


# speedup vs baseline: 502.7618x; 1.0007x over previous
import jax
import jax.numpy as jnp
from jax import lax
from jax.experimental import pallas as pl
from jax.experimental.pallas import tpu as pltpu
from jax.experimental.pallas import tpu_sc as plsc

N = 5000
LANES = 16
NPAD = 5120
NCH = NPAD // LANES
NCLS = 80
HUMAN_IDX = 1
NMS_THRESH = 0.5
SCORE_THRESH = 0.2
KCAP = 15
TILES = 16
CPT = NCLS // TILES
NEGS = -3.0e38
DUMMY = 3.0e9
BIGI = 2**30


def _nms_body(vin, lbh, obh, osh, olh,
              vinf, vlb, rmidx, rmlab, rms, midx, ms, t16f, t16i,
              gsc, gidx, heads_s, heads_i, ptrv, rb, rs, rl,
              sem1, sem2, ssc, sidx):
    core = lax.axis_index("c")
    sub = lax.axis_index("s")
    lanes = lax.iota(jnp.int32, LANES)
    ones = lanes >= 0
    negs16 = jnp.full((LANES,), NEGS, jnp.float32)
    bigi16 = jnp.full((LANES,), BIGI, jnp.int32)

    @pl.when(core == 0)
    def _():
        cp1 = pltpu.async_copy(vin, vinf, sem1)
        cp2 = pltpu.async_copy(lbh, vlb, sem2)
        cp1.wait()
        cp2.wait()

        def mx_body(j, acc):
            a = jnp.maximum(vinf[2, pl.ds(j * 2 * LANES, LANES)],
                            vinf[3, pl.ds(j * 2 * LANES, LANES)])
            b = jnp.maximum(vinf[2, pl.ds(j * 2 * LANES + LANES, LANES)],
                            vinf[3, pl.ds(j * 2 * LANES + LANES, LANES)])
            return jnp.maximum(acc, jnp.maximum(a, b))

        acc = lax.fori_loop(0, NCH // 2, mx_body, negs16)
        maxc = jnp.max(acc) + jnp.float32(1.0)

        lo = sub * CPT

        def rscan(j, cnt):
            lab16 = vlb[pl.ds(j * LANES, LANES)]
            sc16 = vinf[4, pl.ds(j * LANES, LANES)]
            m = (lab16 >= lo) & (lab16 < lo + CPT) & (sc16 >= SCORE_THRESH)
            idx16 = j * LANES + lanes
            plsc.store_compressed(rmidx.at[pl.ds(cnt, LANES)], idx16, mask=m)
            plsc.store_compressed(rmlab.at[pl.ds(cnt, LANES)], lab16, mask=m)
            plsc.store_compressed(rms.at[pl.ds(cnt, LANES)], sc16, mask=m)
            return cnt + plsc.all_reduce_population_count(m)[0]

        rcnt = lax.fori_loop(0, NCH, rscan, jnp.int32(0))
        plsc.store_compressed(rmlab.at[pl.ds(rcnt, LANES)],
                              jnp.full((LANES,), -1, jnp.int32), mask=ones)
        rch = (rcnt + (LANES - 1)) >> 4

        for k in range(CPT):
            c = lo + k
            off = c.astype(jnp.float32) * maxc

            def scan_body(j, cnt):
                lab16 = rmlab[pl.ds(j * LANES, LANES)]
                m = lab16 == c
                plsc.store_compressed(midx.at[pl.ds(cnt, LANES)],
                                      rmidx[pl.ds(j * LANES, LANES)], mask=m)
                plsc.store_compressed(ms.at[pl.ds(cnt, LANES)],
                                      rms[pl.ds(j * LANES, LANES)], mask=m)
                return cnt + plsc.all_reduce_population_count(m)[0]

            cnt = lax.fori_loop(0, rch, scan_body, jnp.int32(0))
            plsc.store_compressed(ms.at[pl.ds(cnt, LANES)], negs16, mask=ones)

            def cond(st):
                return (st[0] < cnt) & (st[1] < KCAP)

            def body(st):
                nproc, kcnt, kx1, ky1, kx2, ky2, kid, ksc = st
                nchk = (cnt + (LANES - 1)) >> 4

                def am_body(j, s):
                    bv, bp = s
                    v = ms[pl.ds(j * LANES, LANES)]
                    upd = v > bv
                    return jnp.where(upd, v, bv), jnp.where(upd, j, bp)

                bv, bp = lax.fori_loop(0, nchk, am_body,
                                       (negs16, jnp.zeros((LANES,), jnp.int32)))
                gmax = jnp.max(bv)
                posl = jnp.where(bv == gmax, bp * LANES + lanes, BIGI)
                pos = jnp.min(posl)
                posv = jnp.full((LANES,), pos, jnp.int32)
                plsc.store_scatter(ms, [posv], negs16, mask=lanes == 0)
                giv = plsc.load_gather(midx, [posv])
                c0 = jnp.zeros((LANES,), jnp.int32)
                cx1 = plsc.load_gather(vinf, [c0, giv]) + off
                cy1 = plsc.load_gather(vinf, [c0 + 1, giv]) + off
                cx2 = plsc.load_gather(vinf, [c0 + 2, giv]) + off
                cy2 = plsc.load_gather(vinf, [c0 + 3, giv]) + off
                w = jnp.maximum(jnp.minimum(kx2, cx2) - jnp.maximum(kx1, cx1), 0.0)
                h = jnp.maximum(jnp.minimum(ky2, cy2) - jnp.maximum(ky1, cy1), 0.0)
                inter = w * h
                ka = (kx2 - kx1) * (ky2 - ky1)
                ca = (cx2 - cx1) * (cy2 - cy1)
                iou = inter / jnp.maximum(ka + ca - inter, jnp.float32(1e-9))
                sup = plsc.all_reduce_population_count(iou > NMS_THRESH)[0] > 0
                addm = jnp.logical_and(jnp.logical_not(sup), lanes == kcnt)
                kx1 = jnp.where(addm, cx1, kx1)
                ky1 = jnp.where(addm, cy1, ky1)
                kx2 = jnp.where(addm, cx2, kx2)
                ky2 = jnp.where(addm, cy2, ky2)
                kid = jnp.where(addm, giv, kid)
                ksc = jnp.where(addm, gmax, ksc)
                kcnt = kcnt + jnp.where(sup, 0, 1).astype(jnp.int32)
                return (nproc + 1, kcnt, kx1, ky1, kx2, ky2, kid, ksc)

            dummy16 = jnp.full((LANES,), DUMMY, jnp.float32)
            st = lax.while_loop(cond, body,
                                (jnp.int32(0), jnp.int32(0),
                                 dummy16, dummy16, dummy16, dummy16,
                                 bigi16, negs16))
            kid, ksc = st[6], st[7]
            t16f[...] = ksc
            pltpu.sync_copy(t16f, ssc.at[pl.ds(c * LANES, LANES)])
            t16i[...] = kid
            pltpu.sync_copy(t16i, sidx.at[pl.ds(c * LANES, LANES)])

        plsc.subcore_barrier()

        @pl.when(sub == 0)
        def _():
            pltpu.sync_copy(ssc, gsc)
            pltpu.sync_copy(sidx, gidx)
            hs = gsc[pl.ds(HUMAN_IDX * LANES, LANES)]
            hi = gidx[pl.ds(HUMAN_IDX * LANES, LANES)]
            gsc[pl.ds(HUMAN_IDX * LANES, LANES)] = negs16
            for j in range(NCLS // LANES):
                rowv = (j * LANES + lanes) * LANES
                heads_s[pl.ds(j * LANES, LANES)] = plsc.load_gather(gsc, [rowv])
                heads_i[pl.ds(j * LANES, LANES)] = plsc.load_gather(gidx, [rowv])
            one16 = jnp.full((LANES,), 1, jnp.int32)
            for j in range(NCLS // LANES):
                ptrv[pl.ds(j * LANES, LANES)] = one16

            def ext_body(t, s):
                osc, oidx = s

                def hb(j, hst):
                    bv, bi, bp = hst
                    v = heads_s[pl.ds(j * LANES, LANES)]
                    iv = heads_i[pl.ds(j * LANES, LANES)]
                    upd = (v > bv) | ((v == bv) & (iv < bi))
                    return (jnp.where(upd, v, bv), jnp.where(upd, iv, bi),
                            jnp.where(upd, j, bp))

                bv, bi, bp = lax.fori_loop(0, NCLS // LANES, hb,
                                           (negs16, bigi16,
                                            jnp.zeros((LANES,), jnp.int32)))
                gmax = jnp.max(bv)
                gidw = jnp.min(jnp.where(bv == gmax, bi, BIGI))
                cls = jnp.min(jnp.where((bv == gmax) & (bi == gidw),
                                        bp * LANES + lanes, BIGI))
                clsv = jnp.full((LANES,), cls, jnp.int32)
                p = plsc.load_gather(ptrv, [clsv])
                plsc.store_scatter(ptrv, [clsv], p + 1, mask=lanes == 0)
                psafe = jnp.minimum(p, LANES - 1)
                hv = plsc.load_gather(gsc, [clsv * LANES + psafe])
                hiv = plsc.load_gather(gidx, [clsv * LANES + psafe])
                hv = jnp.where(p >= LANES, negs16, hv)
                plsc.store_scatter(heads_s, [clsv], hv, mask=lanes == 0)
                plsc.store_scatter(heads_i, [clsv], hiv, mask=lanes == 0)
                valid = gmax > jnp.float32(-1.0e37)
                osc = jnp.where((lanes == t) & valid, gmax, osc)
                oidx = jnp.where((lanes == t) & valid, gidw, oidx)
                return (osc, oidx)

            osc, oidx = lax.fori_loop(0, KCAP, ext_body, (negs16, bigi16))

            heads_s[pl.ds(0, LANES)] = hs
            heads_i[pl.ds(0, LANES)] = hi
            heads_s[pl.ds(LANES, LANES)] = osc
            heads_i[pl.ds(LANES, LANES)] = oidx

            zf16 = jnp.zeros((LANES,), jnp.float32)
            for j in range(8):
                rb[pl.ds(j * LANES, LANES)] = zf16
            rs[pl.ds(0, LANES)] = zf16
            rs[pl.ds(LANES, LANES)] = zf16
            neg1 = jnp.full((LANES,), -1, jnp.int32)
            rl[pl.ds(0, LANES)] = neg1
            rl[pl.ds(LANES, LANES)] = neg1

            def mg_body(t, s):
                hp, op = s
                hpv = jnp.full((LANES,), hp, jnp.int32)
                opv = jnp.full((LANES,), op + LANES, jnp.int32)
                hv = plsc.load_gather(heads_s, [hpv])
                hiv = plsc.load_gather(heads_i, [hpv])
                ov = plsc.load_gather(heads_s, [opv])
                oiv = plsc.load_gather(heads_i, [opv])
                hvs = hv[0]
                ovs = ov[0]
                his = hiv[0]
                ois = oiv[0]
                takeh = (hvs > ovs) | ((hvs == ovs) & (his < ois))
                cs = jnp.where(takeh, hv, ov)
                ci = jnp.where(takeh, hiv, oiv)
                valid = cs[0] > jnp.float32(-1.0e37)
                cis = jnp.where(valid, ci, jnp.zeros((LANES,), jnp.int32))
                c0 = jnp.zeros((LANES,), jnp.int32)
                m0 = (lanes == 0) & valid
                tv = jnp.full((LANES,), t, jnp.int32)
                bx1 = plsc.load_gather(vinf, [c0, cis])
                by1 = plsc.load_gather(vinf, [c0 + 1, cis])
                bx2 = plsc.load_gather(vinf, [c0 + 2, cis])
                by2 = plsc.load_gather(vinf, [c0 + 3, cis])
                lbv = plsc.load_gather(vlb, [cis])
                plsc.store_scatter(rb, [tv * 4], bx1, mask=m0)
                plsc.store_scatter(rb, [tv * 4 + 1], by1, mask=m0)
                plsc.store_scatter(rb, [tv * 4 + 2], bx2, mask=m0)
                plsc.store_scatter(rb, [tv * 4 + 3], by2, mask=m0)
                plsc.store_scatter(rs, [tv], cs, mask=m0)
                plsc.store_scatter(rl, [tv], lbv, mask=m0)
                adv = valid.astype(jnp.int32)
                hp = hp + jnp.where(takeh, adv, 0)
                op = op + jnp.where(takeh, 0, adv)
                return (hp, op)

            lax.fori_loop(0, 2 * KCAP, mg_body, (jnp.int32(0), jnp.int32(0)))

            pltpu.sync_copy(rb, obh)
            pltpu.sync_copy(rs, osh)
            pltpu.sync_copy(rl, olh)


_mesh = plsc.VectorSubcoreMesh(core_axis_name="c", subcore_axis_name="s",
                               num_cores=2, num_subcores=16)

_OUT_TYPE = [
    jax.ShapeDtypeStruct((128,), jnp.float32),
    jax.ShapeDtypeStruct((32,), jnp.float32),
    jax.ShapeDtypeStruct((32,), jnp.int32),
]

_SCRATCH_TYPES = [
        pltpu.VMEM((5, NPAD), jnp.float32),
        pltpu.VMEM((NPAD,), jnp.int32),
        pltpu.VMEM((NPAD + LANES,), jnp.int32),
        pltpu.VMEM((NPAD + LANES,), jnp.int32),
        pltpu.VMEM((NPAD + LANES,), jnp.float32),
        pltpu.VMEM((NPAD + LANES,), jnp.int32),
        pltpu.VMEM((NPAD + LANES,), jnp.float32),
        pltpu.VMEM((LANES,), jnp.float32),
        pltpu.VMEM((LANES,), jnp.int32),
        pltpu.VMEM((NCLS * LANES,), jnp.float32),
        pltpu.VMEM((NCLS * LANES,), jnp.int32),
        pltpu.VMEM((NCLS,), jnp.float32),
        pltpu.VMEM((NCLS,), jnp.int32),
        pltpu.VMEM((NCLS,), jnp.int32),
        pltpu.VMEM((128,), jnp.float32),
        pltpu.VMEM((32,), jnp.float32),
        pltpu.VMEM((32,), jnp.int32),
        pltpu.SemaphoreType.DMA,
        pltpu.SemaphoreType.DMA,
        pltpu.VMEM_SHARED((NCLS * LANES,), jnp.float32),
        pltpu.VMEM_SHARED((NCLS * LANES,), jnp.int32),
]

_sc_call = pl.kernel(
    _nms_body,
    out_type=_OUT_TYPE,
    mesh=_mesh,
    compiler_params=pltpu.CompilerParams(needs_layout_passes=False,
                                         skip_device_barrier=True),
    scratch_types=_SCRATCH_TYPES,
)


@jax.jit
def kernel(boxes, scores, labels):
    pad = NPAD - N
    zf = jnp.zeros((pad,), jnp.float32)
    vin = jnp.stack([
        jnp.concatenate([boxes[:, 0], zf]),
        jnp.concatenate([boxes[:, 1], zf]),
        jnp.concatenate([boxes[:, 2], zf]),
        jnp.concatenate([boxes[:, 3], zf]),
        jnp.concatenate([scores, jnp.full((pad,), -1.0, jnp.float32)]),
    ])
    lb = jnp.concatenate([labels, jnp.full((pad,), -1, jnp.int32)])
    obf, osf, olf = _sc_call(vin, lb)
    return obf[:120].reshape(30, 4), osf[:30], olf[:30]
